# Initial kernel scaffold; baseline (speedup 1.0000x reference)
import functools
import math

import jax
import jax.numpy as jnp
from jax.experimental import pallas as pl
from jax.experimental.pallas import tpu as pltpu

R_MAX = 5.0
AVG_NEIGH = 32.0
N_RADIAL = 8
NUM_SPECIES = 5


def _dense_body(ts_ref, tr_ref, w0s_ref, w0r_ref, w0c_ref, w1_ref,
                wla_ref, wlb_ref, wout_ref, out_ref):
    ts = ts_ref[...]
    tr = tr_ref[...]
    rel = (tr[:, 0:3] - ts[:, 0:3]) * (1.0 / R_MAX)
    r2 = jnp.sum(rel * rel, axis=-1, keepdims=True)
    d2 = r2 + 1e-9
    d = jnp.sqrt(d2)
    xr = jnp.clip(d, 1e-4, 1.0)
    nb = 1.0 + jax.lax.broadcasted_iota(jnp.float32, (1, N_RADIAL), 1)
    cutoff = 0.5 * (jnp.cos(jnp.pi * jnp.clip(d, 0.0, 1.0)) + 1.0)
    rbf = (math.sqrt(2.0) * jnp.sin(nb * (jnp.pi * xr)) / xr) * cutoff

    spec_s = ts[:, 3:4]
    spec_r = tr[:, 3:4]
    h0 = jnp.dot(rbf, w0c_ref[...], preferred_element_type=jnp.float32)
    for c in range(NUM_SPECIES):
        cf = jnp.float32(c)
        h0 = h0 + jnp.where(spec_s == cf, 1.0, 0.0) * w0s_ref[c][None, :]
        h0 = h0 + jnp.where(spec_r == cf, 1.0, 0.0) * w0r_ref[c][None, :]
    h = jax.nn.silu(h0)
    h = jax.nn.silu(jnp.dot(h, w1_ref[...], preferred_element_type=jnp.float32))

    ny = 1.0 + 3.0 * (r2 / d2)
    g = h * 0.5
    for i in range(wla_ref.shape[0]):
        inv2 = (g * g) * ny
        h = jax.nn.silu(
            jnp.dot(h, wla_ref[i], preferred_element_type=jnp.float32)
            + jnp.dot(inv2, wlb_ref[i], preferred_element_type=jnp.float32))
        g = g * h * (1.0 / math.sqrt(AVG_NEIGH))

    out_ref[...] = jnp.dot(h, wout_ref[...],
                           preferred_element_type=jnp.float32) * cutoff


def _edge_dense(ts, tr, W0, W1, Wl, Wout, block_e, interpret=False):
    E = ts.shape[0]
    hidden = W1.shape[0]
    W0s = W0[0:NUM_SPECIES]
    W0r = W0[NUM_SPECIES:2 * NUM_SPECIES]
    W0c = W0[2 * NUM_SPECIES:]
    Wla = Wl[:, :hidden, :]
    Wlb = Wl[:, hidden:, :]
    grid = (E // block_e,)
    full = lambda s: pl.BlockSpec(s, lambda i: tuple(0 for _ in s))
    return pl.pallas_call(
        _dense_body,
        grid=grid,
        in_specs=[
            pl.BlockSpec((block_e, 8), lambda i: (i, 0)),
            pl.BlockSpec((block_e, 8), lambda i: (i, 0)),
            full(W0s.shape), full(W0r.shape), full(W0c.shape),
            full(W1.shape), full(Wla.shape), full(Wlb.shape),
            full(Wout.shape),
        ],
        out_specs=pl.BlockSpec((block_e, Wout.shape[1]), lambda i: (i, 0)),
        out_shape=jax.ShapeDtypeStruct((E, Wout.shape[1]), jnp.float32),
        interpret=interpret,
    )(ts, tr, W0s, W0r, W0c, W1, Wla, Wlb, Wout)


def kernel(positions, species, senders, receivers, W0, W1, Wl, Wout):
    n = positions.shape[0]
    T = jnp.concatenate(
        [positions.astype(jnp.float32),
         species.astype(jnp.float32)[:, None],
         jnp.zeros((n, 4), jnp.float32)], axis=1)
    ts = T[senders]
    tr = T[receivers]
    edge = _edge_dense(ts, tr, W0, W1, Wl, Wout, block_e=3200)
    node = jax.ops.segment_sum(edge, receivers, num_segments=n) \
         + jax.ops.segment_sum(edge, senders, num_segments=n)
    return node


# speedup vs baseline: 1.0257x; 1.0257x over previous
import functools
import math

import jax
import jax.numpy as jnp
from jax.experimental import pallas as pl
from jax.experimental.pallas import tpu as pltpu

R_MAX = 5.0
AVG_NEIGH = 32.0
N_RADIAL = 8
NUM_SPECIES = 5


def _dense_body(ts_ref, tr_ref, w0s_ref, w0r_ref, w0c_ref, w1_ref,
                wla_ref, wlb_ref, wout_ref, out_ref):
    ts = ts_ref[...]
    tr = tr_ref[...]
    rel = (tr[:, 0:3] - ts[:, 0:3]) * (1.0 / R_MAX)
    r2 = jnp.sum(rel * rel, axis=-1, keepdims=True)
    d2 = r2 + 1e-9
    d = jnp.sqrt(d2)
    xr = jnp.clip(d, 1e-4, 1.0)
    nb = 1.0 + jax.lax.broadcasted_iota(
        jnp.int32, (ts.shape[0], N_RADIAL), 1).astype(jnp.float32)
    cutoff = 0.5 * (jnp.cos(jnp.pi * jnp.clip(d, 0.0, 1.0)) + 1.0)
    rbf = (math.sqrt(2.0) * jnp.sin(nb * (jnp.pi * xr)) / xr) * cutoff

    spec_s = ts[:, 3:4]
    spec_r = tr[:, 3:4]
    h0 = jnp.dot(rbf, w0c_ref[...], preferred_element_type=jnp.float32)
    for c in range(NUM_SPECIES):
        cf = jnp.float32(c)
        h0 = h0 + jnp.where(spec_s == cf, 1.0, 0.0) * w0s_ref[c][None, :]
        h0 = h0 + jnp.where(spec_r == cf, 1.0, 0.0) * w0r_ref[c][None, :]
    h = jax.nn.silu(h0)
    h = jax.nn.silu(jnp.dot(h, w1_ref[...], preferred_element_type=jnp.float32))

    ny = 1.0 + 3.0 * (r2 / d2)
    g = h * 0.5
    for i in range(wla_ref.shape[0]):
        inv2 = (g * g) * ny
        h = jax.nn.silu(
            jnp.dot(h, wla_ref[i], preferred_element_type=jnp.float32)
            + jnp.dot(inv2, wlb_ref[i], preferred_element_type=jnp.float32))
        g = g * h * (1.0 / math.sqrt(AVG_NEIGH))

    out_ref[...] = jnp.dot(h, wout_ref[...],
                           preferred_element_type=jnp.float32) * cutoff


def _edge_dense(ts, tr, W0, W1, Wl, Wout, block_e, interpret=False):
    E = ts.shape[0]
    hidden = W1.shape[0]
    W0s = W0[0:NUM_SPECIES]
    W0r = W0[NUM_SPECIES:2 * NUM_SPECIES]
    W0c = W0[2 * NUM_SPECIES:]
    Wla = Wl[:, :hidden, :]
    Wlb = Wl[:, hidden:, :]
    grid = (E // block_e,)
    full = lambda s: pl.BlockSpec(s, lambda i: tuple(0 for _ in s))
    return pl.pallas_call(
        _dense_body,
        grid=grid,
        in_specs=[
            pl.BlockSpec((block_e, 8), lambda i: (i, 0)),
            pl.BlockSpec((block_e, 8), lambda i: (i, 0)),
            full(W0s.shape), full(W0r.shape), full(W0c.shape),
            full(W1.shape), full(Wla.shape), full(Wlb.shape),
            full(Wout.shape),
        ],
        out_specs=pl.BlockSpec((block_e, Wout.shape[1]), lambda i: (i, 0)),
        out_shape=jax.ShapeDtypeStruct((E, Wout.shape[1]), jnp.float32),
        interpret=interpret,
    )(ts, tr, W0s, W0r, W0c, W1, Wla, Wlb, Wout)


def kernel(positions, species, senders, receivers, W0, W1, Wl, Wout):
    n = positions.shape[0]
    T = jnp.concatenate(
        [positions.astype(jnp.float32),
         species.astype(jnp.float32)[:, None],
         jnp.zeros((n, 4), jnp.float32)], axis=1)
    ts = T[senders]
    tr = T[receivers]
    edge = _edge_dense(ts, tr, W0, W1, Wl, Wout, block_e=3200)
    node = jax.ops.segment_sum(edge, receivers, num_segments=n) \
         + jax.ops.segment_sum(edge, senders, num_segments=n)
    return node


# speedup vs baseline: 1.3497x; 1.3160x over previous
import functools
import math

import jax
import jax.numpy as jnp
from jax import lax
from jax.experimental import pallas as pl
from jax.experimental.pallas import tpu as pltpu
from jax.experimental.pallas import tpu_sc as plsc

R_MAX = 5.0
AVG_NEIGH = 32.0
N_RADIAL = 8
NUM_SPECIES = 5

NC = 2
NS = 16
NW = NC * NS


def _sc_gather(T, senders, receivers):
    n, d = T.shape
    e = senders.shape[0]
    ew = e // NW
    mesh = plsc.VectorSubcoreMesh(core_axis_name="c", subcore_axis_name="s")

    @functools.partial(
        pl.kernel,
        out_type=(jax.ShapeDtypeStruct((e, d), jnp.float32),
                  jax.ShapeDtypeStruct((e, d), jnp.float32)),
        mesh=mesh,
        scratch_types=[
            pltpu.VMEM((ew,), jnp.int32),
            pltpu.VMEM((ew, d), jnp.float32),
            pltpu.SemaphoreType.DMA,
        ],
        compiler_params=pltpu.CompilerParams(use_tc_tiling_on_sc=False),
    )
    def k(t_hbm, snd_hbm, rcv_hbm, outs_hbm, outr_hbm, idx_v, rows_v, sem):
        wid = lax.axis_index("s") * NC + lax.axis_index("c")
        base = wid * ew
        pltpu.sync_copy(snd_hbm.at[pl.ds(base, ew)], idx_v)
        pltpu.async_copy(t_hbm.at[idx_v], rows_v, sem).wait()
        pltpu.sync_copy(rows_v, outs_hbm.at[pl.ds(base, ew)])
        pltpu.sync_copy(rcv_hbm.at[pl.ds(base, ew)], idx_v)
        pltpu.async_copy(t_hbm.at[idx_v], rows_v, sem).wait()
        pltpu.sync_copy(rows_v, outr_hbm.at[pl.ds(base, ew)])

    return k(T, senders, receivers)


def _dense_body(ts_ref, tr_ref, w0s_ref, w0r_ref, w0c_ref, w1_ref,
                wla_ref, wlb_ref, wout_ref, out_ref):
    ts = ts_ref[...]
    tr = tr_ref[...]
    rel = (tr[:, 0:3] - ts[:, 0:3]) * (1.0 / R_MAX)
    r2 = jnp.sum(rel * rel, axis=-1, keepdims=True)
    d2 = r2 + 1e-9
    d = jnp.sqrt(d2)
    xr = jnp.clip(d, 1e-4, 1.0)
    nb = 1.0 + jax.lax.broadcasted_iota(
        jnp.int32, (ts.shape[0], N_RADIAL), 1).astype(jnp.float32)
    cutoff = 0.5 * (jnp.cos(jnp.pi * jnp.clip(d, 0.0, 1.0)) + 1.0)
    rbf = (math.sqrt(2.0) * jnp.sin(nb * (jnp.pi * xr)) / xr) * cutoff

    spec_s = ts[:, 3:4]
    spec_r = tr[:, 3:4]
    h0 = jnp.dot(rbf, w0c_ref[...], preferred_element_type=jnp.float32)
    for c in range(NUM_SPECIES):
        cf = jnp.float32(c)
        h0 = h0 + jnp.where(spec_s == cf, 1.0, 0.0) * w0s_ref[c][None, :]
        h0 = h0 + jnp.where(spec_r == cf, 1.0, 0.0) * w0r_ref[c][None, :]
    h = jax.nn.silu(h0)
    h = jax.nn.silu(jnp.dot(h, w1_ref[...], preferred_element_type=jnp.float32))

    ny = 1.0 + 3.0 * (r2 / d2)
    g = h * 0.5
    for i in range(wla_ref.shape[0]):
        inv2 = (g * g) * ny
        h = jax.nn.silu(
            jnp.dot(h, wla_ref[i], preferred_element_type=jnp.float32)
            + jnp.dot(inv2, wlb_ref[i], preferred_element_type=jnp.float32))
        g = g * h * (1.0 / math.sqrt(AVG_NEIGH))

    out_ref[...] = jnp.dot(h, wout_ref[...],
                           preferred_element_type=jnp.float32) * cutoff


def _edge_dense(ts, tr, W0, W1, Wl, Wout, block_e, interpret=False):
    E = ts.shape[0]
    hidden = W1.shape[0]
    W0s = W0[0:NUM_SPECIES]
    W0r = W0[NUM_SPECIES:2 * NUM_SPECIES]
    W0c = W0[2 * NUM_SPECIES:]
    Wla = Wl[:, :hidden, :]
    Wlb = Wl[:, hidden:, :]
    grid = (E // block_e,)
    full = lambda s: pl.BlockSpec(s, lambda i: tuple(0 for _ in s))
    return pl.pallas_call(
        _dense_body,
        grid=grid,
        in_specs=[
            pl.BlockSpec((block_e, 8), lambda i: (i, 0)),
            pl.BlockSpec((block_e, 8), lambda i: (i, 0)),
            full(W0s.shape), full(W0r.shape), full(W0c.shape),
            full(W1.shape), full(Wla.shape), full(Wlb.shape),
            full(Wout.shape),
        ],
        out_specs=pl.BlockSpec((block_e, Wout.shape[1]), lambda i: (i, 0)),
        out_shape=jax.ShapeDtypeStruct((E, Wout.shape[1]), jnp.float32),
        interpret=interpret,
    )(ts, tr, W0s, W0r, W0c, W1, Wla, Wlb, Wout)


def kernel(positions, species, senders, receivers, W0, W1, Wl, Wout):
    n = positions.shape[0]
    T = jnp.concatenate(
        [positions.astype(jnp.float32),
         species.astype(jnp.float32)[:, None],
         jnp.zeros((n, 4), jnp.float32)], axis=1)
    ts, tr = _sc_gather(T, senders.astype(jnp.int32),
                        receivers.astype(jnp.int32))
    edge = _edge_dense(ts, tr, W0, W1, Wl, Wout, block_e=3200)
    node = jax.ops.segment_sum(edge, receivers, num_segments=n) \
         + jax.ops.segment_sum(edge, senders, num_segments=n)
    return node


# speedup vs baseline: 2.2272x; 1.6501x over previous
import functools
import math

import jax
import jax.numpy as jnp
from jax import lax
from jax.experimental import pallas as pl
from jax.experimental.pallas import tpu as pltpu
from jax.experimental.pallas import tpu_sc as plsc

R_MAX = 5.0
AVG_NEIGH = 32.0
N_RADIAL = 8
NUM_SPECIES = 5

NC = 2
NS = 16
NW = NC * NS


def _sc_gather(T, senders, receivers):
    n, d = T.shape
    e = senders.shape[0]
    ew = e // NW
    mesh = plsc.VectorSubcoreMesh(core_axis_name="c", subcore_axis_name="s")

    @functools.partial(
        pl.kernel,
        out_type=(jax.ShapeDtypeStruct((e, d), jnp.float32),
                  jax.ShapeDtypeStruct((e, d), jnp.float32)),
        mesh=mesh,
        scratch_types=[
            pltpu.VMEM((ew,), jnp.int32),
            pltpu.VMEM((ew, d), jnp.float32),
            pltpu.SemaphoreType.DMA,
        ],
        compiler_params=pltpu.CompilerParams(use_tc_tiling_on_sc=False),
    )
    def k(t_hbm, snd_hbm, rcv_hbm, outs_hbm, outr_hbm, idx_v, rows_v, sem):
        wid = lax.axis_index("s") * NC + lax.axis_index("c")
        base = wid * ew
        pltpu.sync_copy(snd_hbm.at[pl.ds(base, ew)], idx_v)
        pltpu.async_copy(t_hbm.at[idx_v], rows_v, sem).wait()
        pltpu.sync_copy(rows_v, outs_hbm.at[pl.ds(base, ew)])
        pltpu.sync_copy(rcv_hbm.at[pl.ds(base, ew)], idx_v)
        pltpu.async_copy(t_hbm.at[idx_v], rows_v, sem).wait()
        pltpu.sync_copy(rows_v, outr_hbm.at[pl.ds(base, ew)])

    return k(T, senders, receivers)


def _sc_scatter(edge, snd2, rcv2, zeros):
    e, dout = edge.shape
    n = zeros.shape[0]
    nchunk, ch = snd2.shape
    ew = e // NW
    cw = nchunk // NW
    eb = 2000
    nblk = ew // eb
    cpb = eb // ch
    nslice = n // NS
    mesh = plsc.VectorSubcoreMesh(core_axis_name="c", subcore_axis_name="s")

    @functools.partial(
        pl.kernel,
        out_type=jax.ShapeDtypeStruct((NC * n, dout), jnp.float32),
        mesh=mesh,
        scratch_types=[
            pltpu.VMEM_SHARED((n, dout), jnp.float32),
            pltpu.VMEM((cw, ch), jnp.int32),
            pltpu.VMEM((cw, ch), jnp.int32),
            pltpu.VMEM((eb, dout), jnp.float32),
        ],
        compiler_params=pltpu.CompilerParams(use_tc_tiling_on_sc=False),
    )
    def k(edge_hbm, snd_hbm, rcv_hbm, z_hbm, out_hbm, acc, sidx, ridx, ebuf):
        cid = lax.axis_index("c")
        sid = lax.axis_index("s")
        wid = sid * NC + cid
        pltpu.sync_copy(z_hbm.at[pl.ds(sid * nslice, nslice)],
                        acc.at[pl.ds(sid * nslice, nslice)])
        pltpu.sync_copy(snd_hbm.at[pl.ds(wid * cw, cw)], sidx)
        pltpu.sync_copy(rcv_hbm.at[pl.ds(wid * cw, cw)], ridx)
        plsc.subcore_barrier()
        for blk in range(nblk):
            pltpu.sync_copy(edge_hbm.at[pl.ds(wid * ew + blk * eb, eb)], ebuf)
            for j in range(cpb):
                c = blk * cpb + j
                sl = ebuf.at[pl.ds(j * ch, ch)]
                pltpu.sync_copy(sl, acc.at[ridx.at[c]], add=True)
                pltpu.sync_copy(sl, acc.at[sidx.at[c]], add=True)
        plsc.subcore_barrier()
        pltpu.sync_copy(acc.at[pl.ds(sid * nslice, nslice)],
                        out_hbm.at[pl.ds(cid * n + sid * nslice, nslice)])

    return k(edge, snd2, rcv2, zeros)


def _dense_body(ts_ref, tr_ref, w0s_ref, w0r_ref, w0c_ref, w1_ref,
                wla_ref, wlb_ref, wout_ref, out_ref):
    ts = ts_ref[...]
    tr = tr_ref[...]
    rel = (tr[:, 0:3] - ts[:, 0:3]) * (1.0 / R_MAX)
    r2 = jnp.sum(rel * rel, axis=-1, keepdims=True)
    d2 = r2 + 1e-9
    d = jnp.sqrt(d2)
    xr = jnp.clip(d, 1e-4, 1.0)
    nb = 1.0 + jax.lax.broadcasted_iota(
        jnp.int32, (ts.shape[0], N_RADIAL), 1).astype(jnp.float32)
    cutoff = 0.5 * (jnp.cos(jnp.pi * jnp.clip(d, 0.0, 1.0)) + 1.0)
    rbf = (math.sqrt(2.0) * jnp.sin(nb * (jnp.pi * xr)) / xr) * cutoff

    spec_s = ts[:, 3:4]
    spec_r = tr[:, 3:4]
    h0 = jnp.dot(rbf, w0c_ref[...], preferred_element_type=jnp.float32)
    for c in range(NUM_SPECIES):
        cf = jnp.float32(c)
        h0 = h0 + jnp.where(spec_s == cf, 1.0, 0.0) * w0s_ref[c][None, :]
        h0 = h0 + jnp.where(spec_r == cf, 1.0, 0.0) * w0r_ref[c][None, :]
    h = jax.nn.silu(h0)
    h = jax.nn.silu(jnp.dot(h, w1_ref[...], preferred_element_type=jnp.float32))

    ny = 1.0 + 3.0 * (r2 / d2)
    g = h * 0.5
    for i in range(wla_ref.shape[0]):
        inv2 = (g * g) * ny
        h = jax.nn.silu(
            jnp.dot(h, wla_ref[i], preferred_element_type=jnp.float32)
            + jnp.dot(inv2, wlb_ref[i], preferred_element_type=jnp.float32))
        g = g * h * (1.0 / math.sqrt(AVG_NEIGH))

    out_ref[...] = jnp.dot(h, wout_ref[...],
                           preferred_element_type=jnp.float32) * cutoff


def _edge_dense(ts, tr, W0, W1, Wl, Wout, block_e, interpret=False):
    E = ts.shape[0]
    hidden = W1.shape[0]
    W0s = W0[0:NUM_SPECIES]
    W0r = W0[NUM_SPECIES:2 * NUM_SPECIES]
    W0c = W0[2 * NUM_SPECIES:]
    Wla = Wl[:, :hidden, :]
    Wlb = Wl[:, hidden:, :]
    grid = (E // block_e,)
    full = lambda s: pl.BlockSpec(s, lambda i: tuple(0 for _ in s))
    return pl.pallas_call(
        _dense_body,
        grid=grid,
        in_specs=[
            pl.BlockSpec((block_e, 8), lambda i: (i, 0)),
            pl.BlockSpec((block_e, 8), lambda i: (i, 0)),
            full(W0s.shape), full(W0r.shape), full(W0c.shape),
            full(W1.shape), full(Wla.shape), full(Wlb.shape),
            full(Wout.shape),
        ],
        out_specs=pl.BlockSpec((block_e, Wout.shape[1]), lambda i: (i, 0)),
        out_shape=jax.ShapeDtypeStruct((E, Wout.shape[1]), jnp.float32),
        interpret=interpret,
    )(ts, tr, W0s, W0r, W0c, W1, Wla, Wlb, Wout)


def kernel(positions, species, senders, receivers, W0, W1, Wl, Wout):
    n = positions.shape[0]
    T = jnp.concatenate(
        [positions.astype(jnp.float32),
         species.astype(jnp.float32)[:, None],
         jnp.zeros((n, 4), jnp.float32)], axis=1)
    ts, tr = _sc_gather(T, senders.astype(jnp.int32),
                        receivers.astype(jnp.int32))
    edge = _edge_dense(ts, tr, W0, W1, Wl, Wout, block_e=3200)
    e = senders.shape[0]
    ch = 125
    snd2 = senders.astype(jnp.int32).reshape(e // ch, ch)
    rcv2 = receivers.astype(jnp.int32).reshape(e // ch, ch)
    part = _sc_scatter(edge, snd2, rcv2,
                       jnp.zeros((n, edge.shape[1]), jnp.float32))
    return part[:n] + part[n:]


# speedup vs baseline: 6.7593x; 3.0349x over previous
import functools
import math

import jax
import jax.numpy as jnp
from jax import lax
from jax.experimental import pallas as pl
from jax.experimental.pallas import tpu as pltpu
from jax.experimental.pallas import tpu_sc as plsc

R_MAX = 5.0
AVG_NEIGH = 32.0
N_RADIAL = 8
NUM_SPECIES = 5

NC = 2
NS = 16
NW = NC * NS


def _sc_gather(T, senders, receivers):
    n, d = T.shape
    e = senders.shape[0]
    ew = e // NW
    mesh = plsc.VectorSubcoreMesh(core_axis_name="c", subcore_axis_name="s")

    @functools.partial(
        pl.kernel,
        out_type=(jax.ShapeDtypeStruct((e, d), jnp.float32),
                  jax.ShapeDtypeStruct((e, d), jnp.float32)),
        mesh=mesh,
        scratch_types=[
            pltpu.VMEM((ew,), jnp.int32),
            pltpu.VMEM((ew, d), jnp.float32),
            pltpu.SemaphoreType.DMA,
        ],
        compiler_params=pltpu.CompilerParams(use_tc_tiling_on_sc=False),
    )
    def k(t_hbm, snd_hbm, rcv_hbm, outs_hbm, outr_hbm, idx_v, rows_v, sem):
        wid = lax.axis_index("s") * NC + lax.axis_index("c")
        base = wid * ew
        pltpu.sync_copy(snd_hbm.at[pl.ds(base, ew)], idx_v)
        pltpu.async_copy(t_hbm.at[idx_v], rows_v, sem).wait()
        pltpu.sync_copy(rows_v, outs_hbm.at[pl.ds(base, ew)])
        pltpu.sync_copy(rcv_hbm.at[pl.ds(base, ew)], idx_v)
        pltpu.async_copy(t_hbm.at[idx_v], rows_v, sem).wait()
        pltpu.sync_copy(rows_v, outr_hbm.at[pl.ds(base, ew)])

    return k(T, senders, receivers)


def _sc_scatter(edge, snd2, rcv2, zeros):
    e, dout = edge.shape
    n = zeros.shape[0]
    nchunk, ch = snd2.shape
    ew = e // NW
    cw = nchunk // NW
    eb = 2000
    nblk = ew // eb
    cpb = eb // ch
    nslice = n // NS
    mesh = plsc.VectorSubcoreMesh(core_axis_name="c", subcore_axis_name="s")

    @functools.partial(
        pl.kernel,
        out_type=jax.ShapeDtypeStruct((NC * n, dout), jnp.float32),
        mesh=mesh,
        scratch_types=[
            pltpu.VMEM_SHARED((n, dout), jnp.float32),
            pltpu.VMEM((cw, ch), jnp.int32),
            pltpu.VMEM((cw, ch), jnp.int32),
            pltpu.VMEM((eb, dout), jnp.float32),
        ],
        compiler_params=pltpu.CompilerParams(use_tc_tiling_on_sc=False),
    )
    def k(edge_hbm, snd_hbm, rcv_hbm, z_hbm, out_hbm, acc, sidx, ridx, ebuf):
        cid = lax.axis_index("c")
        sid = lax.axis_index("s")
        wid = sid * NC + cid
        pltpu.sync_copy(z_hbm.at[pl.ds(sid * nslice, nslice)],
                        acc.at[pl.ds(sid * nslice, nslice)])
        pltpu.sync_copy(snd_hbm.at[pl.ds(wid * cw, cw)], sidx)
        pltpu.sync_copy(rcv_hbm.at[pl.ds(wid * cw, cw)], ridx)
        plsc.subcore_barrier()
        for blk in range(nblk):
            pltpu.sync_copy(edge_hbm.at[pl.ds(wid * ew + blk * eb, eb)], ebuf)
            for j in range(cpb):
                c = blk * cpb + j
                sl = ebuf.at[pl.ds(j * ch, ch)]
                pltpu.sync_copy(sl, acc.at[ridx.at[c]], add=True)
                pltpu.sync_copy(sl, acc.at[sidx.at[c]], add=True)
        plsc.subcore_barrier()
        pltpu.sync_copy(acc.at[pl.ds(sid * nslice, nslice)],
                        out_hbm.at[pl.ds(cid * n + sid * nslice, nslice)])

    return k(edge, snd2, rcv2, zeros)


def _sinpoly(r):
    r2 = r * r
    return r * (1.0 + r2 * (-1.0 / 6.0 + r2 * (1.0 / 120.0
               + r2 * (-1.0 / 5040.0 + r2 * (1.0 / 362880.0)))))


def _dense_body(ts_ref, tr_ref, w0sT_ref, w0rT_ref, w0cT_ref, w1T_ref,
                wlaT_ref, wlbT_ref, woutT_ref, out_ref):
    tsT = jnp.transpose(ts_ref[...])
    trT = jnp.transpose(tr_ref[...])
    b = tsT.shape[1]
    relT = (trT[0:3] - tsT[0:3]) * (1.0 / R_MAX)
    r2 = jnp.sum(relT * relT, axis=0, keepdims=True)
    d2 = r2 + 1e-9
    d = jnp.sqrt(d2)
    xr = jnp.clip(d, 1e-4, 1.0)
    t = jnp.clip(d, 0.0, 1.0) - 0.5
    cutoff = 0.5 - 0.5 * _sinpoly(jnp.pi * t)
    nb = 1.0 + jax.lax.broadcasted_iota(
        jnp.int32, (N_RADIAL, b), 0).astype(jnp.float32)
    z = nb * (jnp.pi * xr)
    q = jnp.floor(z * (1.0 / jnp.pi) + 0.5)
    r = z - q * jnp.pi
    par = q * 0.5 - jnp.floor(q * 0.5)
    sign = 1.0 - 4.0 * par
    s = sign * _sinpoly(r)
    rbfT = (math.sqrt(2.0) * s) * (cutoff / xr)

    iota5 = jax.lax.broadcasted_iota(
        jnp.int32, (NUM_SPECIES, b), 0).astype(jnp.float32)
    ohsT = jnp.where(tsT[3:4] == iota5, 1.0, 0.0)
    ohrT = jnp.where(trT[3:4] == iota5, 1.0, 0.0)
    dot = lambda a, x: jnp.dot(a, x, preferred_element_type=jnp.float32)
    h = jax.nn.silu(dot(w0cT_ref[...], rbfT) + dot(w0sT_ref[...], ohsT)
                    + dot(w0rT_ref[...], ohrT))
    h = jax.nn.silu(dot(w1T_ref[...], h))

    ny = 1.0 + 3.0 * (r2 / d2)
    g = h * 0.5
    for i in range(wlaT_ref.shape[0]):
        inv2 = (g * g) * ny
        h = jax.nn.silu(dot(wlaT_ref[i], h) + dot(wlbT_ref[i], inv2))
        g = g * h * (1.0 / math.sqrt(AVG_NEIGH))

    out_ref[...] = jnp.transpose(dot(woutT_ref[...], h) * cutoff)


def _edge_dense(ts, tr, W0, W1, Wl, Wout, block_e, interpret=False):
    E = ts.shape[0]
    hidden = W1.shape[0]
    W0sT = W0[0:NUM_SPECIES].T
    W0rT = W0[NUM_SPECIES:2 * NUM_SPECIES].T
    W0cT = W0[2 * NUM_SPECIES:].T
    WlaT = jnp.swapaxes(Wl[:, :hidden, :], 1, 2)
    WlbT = jnp.swapaxes(Wl[:, hidden:, :], 1, 2)
    WoutT = Wout.T
    grid = (E // block_e,)
    full = lambda s: pl.BlockSpec(s, lambda i: tuple(0 for _ in s))
    return pl.pallas_call(
        _dense_body,
        grid=grid,
        in_specs=[
            pl.BlockSpec((block_e, 8), lambda i: (i, 0)),
            pl.BlockSpec((block_e, 8), lambda i: (i, 0)),
            full(W0sT.shape), full(W0rT.shape), full(W0cT.shape),
            full(W1.T.shape), full(WlaT.shape), full(WlbT.shape),
            full(WoutT.shape),
        ],
        out_specs=pl.BlockSpec((block_e, Wout.shape[1]), lambda i: (i, 0)),
        out_shape=jax.ShapeDtypeStruct((E, Wout.shape[1]), jnp.float32),
        interpret=interpret,
    )(ts, tr, W0sT, W0rT, W0cT, W1.T, WlaT, WlbT, WoutT)


def kernel(positions, species, senders, receivers, W0, W1, Wl, Wout):
    n = positions.shape[0]
    T = jnp.concatenate(
        [positions.astype(jnp.float32),
         species.astype(jnp.float32)[:, None],
         jnp.zeros((n, 4), jnp.float32)], axis=1)
    ts, tr = _sc_gather(T, senders.astype(jnp.int32),
                        receivers.astype(jnp.int32))
    edge = _edge_dense(ts, tr, W0, W1, Wl, Wout, block_e=3200)
    e = senders.shape[0]
    ch = 125
    snd2 = senders.astype(jnp.int32).reshape(e // ch, ch)
    rcv2 = receivers.astype(jnp.int32).reshape(e // ch, ch)
    part = _sc_scatter(edge, snd2, rcv2,
                       jnp.zeros((n, edge.shape[1]), jnp.float32))
    return part[:n] + part[n:]


# speedup vs baseline: 9.4868x; 1.4035x over previous
import functools
import math

import jax
import jax.numpy as jnp
from jax import lax
from jax.experimental import pallas as pl
from jax.experimental.pallas import tpu as pltpu
from jax.experimental.pallas import tpu_sc as plsc

R_MAX = 5.0
AVG_NEIGH = 32.0
N_RADIAL = 8
NUM_SPECIES = 5

NC = 2
NS = 16
NW = NC * NS
BE = 2048


def _gather_perm(a):
    return a.reshape(-1, 16, 128).swapaxes(1, 2).reshape(-1)


def _scatter_perm(a):
    return a.reshape(-1, 4, 512).swapaxes(1, 2).reshape(-1)


def _sc_gather(T, senders, receivers):
    n, d = T.shape
    e = senders.shape[0]
    ew = e // NW
    mesh = plsc.VectorSubcoreMesh(core_axis_name="c", subcore_axis_name="s")

    @functools.partial(
        pl.kernel,
        out_type=(jax.ShapeDtypeStruct((e, d), jnp.float32),
                  jax.ShapeDtypeStruct((e, d), jnp.float32)),
        mesh=mesh,
        scratch_types=[
            pltpu.VMEM((ew,), jnp.int32),
            pltpu.VMEM((ew, d), jnp.float32),
            pltpu.SemaphoreType.DMA,
        ],
        compiler_params=pltpu.CompilerParams(use_tc_tiling_on_sc=False),
    )
    def k(t_hbm, snd_hbm, rcv_hbm, outs_hbm, outr_hbm, idx_v, rows_v, sem):
        wid = lax.axis_index("s") * NC + lax.axis_index("c")
        base = wid * ew
        pltpu.sync_copy(snd_hbm.at[pl.ds(base, ew)], idx_v)
        pltpu.async_copy(t_hbm.at[idx_v], rows_v, sem).wait()
        pltpu.sync_copy(rows_v, outs_hbm.at[pl.ds(base, ew)])
        pltpu.sync_copy(rcv_hbm.at[pl.ds(base, ew)], idx_v)
        pltpu.async_copy(t_hbm.at[idx_v], rows_v, sem).wait()
        pltpu.sync_copy(rows_v, outr_hbm.at[pl.ds(base, ew)])

    return k(T, senders, receivers)


def _sc_scatter(edge, snd2, rcv2, zeros):
    e, dout = edge.shape
    nacc = zeros.shape[0]
    nchunk, ch = snd2.shape
    ew = e // NW
    cw = nchunk // NW
    eb = 2048
    nblk = ew // eb
    cpb = eb // ch
    nslice = nacc // NS
    mesh = plsc.VectorSubcoreMesh(core_axis_name="c", subcore_axis_name="s")

    @functools.partial(
        pl.kernel,
        out_type=jax.ShapeDtypeStruct((NC * nacc, dout), jnp.float32),
        mesh=mesh,
        scratch_types=[
            pltpu.VMEM_SHARED((nacc, dout), jnp.float32),
            pltpu.VMEM((cw, ch), jnp.int32),
            pltpu.VMEM((cw, ch), jnp.int32),
            pltpu.VMEM((eb, dout), jnp.float32),
        ],
        compiler_params=pltpu.CompilerParams(use_tc_tiling_on_sc=False),
    )
    def k(edge_hbm, snd_hbm, rcv_hbm, z_hbm, out_hbm, acc, sidx, ridx, ebuf):
        cid = lax.axis_index("c")
        sid = lax.axis_index("s")
        wid = sid * NC + cid
        pltpu.sync_copy(z_hbm.at[pl.ds(sid * nslice, nslice)],
                        acc.at[pl.ds(sid * nslice, nslice)])
        pltpu.sync_copy(snd_hbm.at[pl.ds(wid * cw, cw)], sidx)
        pltpu.sync_copy(rcv_hbm.at[pl.ds(wid * cw, cw)], ridx)
        plsc.subcore_barrier()
        for blk in range(nblk):
            pltpu.sync_copy(edge_hbm.at[pl.ds(wid * ew + blk * eb, eb)], ebuf)
            for j in range(cpb):
                c = blk * cpb + j
                sl = ebuf.at[pl.ds(j * ch, ch)]
                pltpu.sync_copy(sl, acc.at[ridx.at[c]], add=True)
                pltpu.sync_copy(sl, acc.at[sidx.at[c]], add=True)
        plsc.subcore_barrier()
        pltpu.sync_copy(acc.at[pl.ds(sid * nslice, nslice)],
                        out_hbm.at[pl.ds(cid * nacc + sid * nslice, nslice)])

    return k(edge, snd2, rcv2, zeros)


def _sinpoly(r):
    r2 = r * r
    return r * (1.0 + r2 * (-1.0 / 6.0 + r2 * (1.0 / 120.0
               + r2 * (-1.0 / 5040.0 + r2 * (1.0 / 362880.0)))))


def _dense_body(ts_ref, tr_ref, w0sT_ref, w0rT_ref, w0cT_ref, w1T_ref,
                wlaT_ref, wlbT_ref, woutT_ref, out_ref):
    xs = jnp.transpose(ts_ref[...])
    xr_t = jnp.transpose(tr_ref[...])
    tsT = jnp.concatenate([xs[8 * j:8 * j + 8, :] for j in range(16)],
                          axis=1)
    trT = jnp.concatenate([xr_t[8 * j:8 * j + 8, :] for j in range(16)],
                          axis=1)
    b = tsT.shape[1]
    relT = (trT[0:3] - tsT[0:3]) * (1.0 / R_MAX)
    r2 = jnp.sum(relT * relT, axis=0, keepdims=True)
    d2 = r2 + 1e-9
    d = jnp.sqrt(d2)
    xr = jnp.clip(d, 1e-4, 1.0)
    t = jnp.clip(d, 0.0, 1.0) - 0.5
    cutoff = 0.5 - 0.5 * _sinpoly(jnp.pi * t)
    nb = 1.0 + jax.lax.broadcasted_iota(
        jnp.int32, (N_RADIAL, b), 0).astype(jnp.float32)
    z = nb * (jnp.pi * xr)
    q = jnp.floor(z * (1.0 / jnp.pi) + 0.5)
    r = z - q * jnp.pi
    par = q * 0.5 - jnp.floor(q * 0.5)
    sign = 1.0 - 4.0 * par
    s = sign * _sinpoly(r)
    rbfT = (math.sqrt(2.0) * s) * (cutoff / xr)

    iota5 = jax.lax.broadcasted_iota(
        jnp.int32, (NUM_SPECIES, b), 0).astype(jnp.float32)
    ohsT = jnp.where(tsT[3:4] == iota5, 1.0, 0.0)
    ohrT = jnp.where(trT[3:4] == iota5, 1.0, 0.0)
    dot = lambda a, x: jnp.dot(a, x, preferred_element_type=jnp.float32)
    h = jax.nn.silu(dot(w0cT_ref[...], rbfT) + dot(w0sT_ref[...], ohsT)
                    + dot(w0rT_ref[...], ohrT))
    h = jax.nn.silu(dot(w1T_ref[...], h))

    ny = 1.0 + 3.0 * (r2 / d2)
    g = h * 0.5
    for i in range(wlaT_ref.shape[0]):
        inv2 = (g * g) * ny
        h = jax.nn.silu(dot(wlaT_ref[i], h) + dot(wlbT_ref[i], inv2))
        g = g * h * (1.0 / math.sqrt(AVG_NEIGH))

    eT = dot(woutT_ref[...], h) * cutoff
    y = jnp.concatenate([eT[:, 512 * k:512 * (k + 1)] for k in range(4)],
                        axis=0)
    out_ref[...] = jnp.transpose(y)


def _edge_dense(ts128, tr128, W0, W1, Wl, Wout, interpret=False):
    E = ts128.shape[0] * 16
    hidden = W1.shape[0]
    W0sT = W0[0:NUM_SPECIES].T
    W0rT = W0[NUM_SPECIES:2 * NUM_SPECIES].T
    W0cT = W0[2 * NUM_SPECIES:].T
    WlaT = jnp.swapaxes(Wl[:, :hidden, :], 1, 2)
    WlbT = jnp.swapaxes(Wl[:, hidden:, :], 1, 2)
    WoutT = Wout.T
    grid = (E // BE,)
    full = lambda s: pl.BlockSpec(s, lambda i: tuple(0 for _ in s))
    return pl.pallas_call(
        _dense_body,
        grid=grid,
        in_specs=[
            pl.BlockSpec((BE // 16, 128), lambda i: (i, 0)),
            pl.BlockSpec((BE // 16, 128), lambda i: (i, 0)),
            full(W0sT.shape), full(W0rT.shape), full(W0cT.shape),
            full(W1.T.shape), full(WlaT.shape), full(WlbT.shape),
            full(WoutT.shape),
        ],
        out_specs=pl.BlockSpec((BE // 4, 128), lambda i: (i, 0)),
        out_shape=jax.ShapeDtypeStruct((E // 4, 128), jnp.float32),
        interpret=interpret,
    )(ts128, tr128, W0sT, W0rT, W0cT, W1.T, WlaT, WlbT, WoutT)


def kernel(positions, species, senders, receivers, W0, W1, Wl, Wout):
    n = positions.shape[0]
    e = senders.shape[0]
    epad = -(-e // (NW * BE)) * (NW * BE)
    nacc = -(-(n + 1) // NS) * NS
    snd = senders.astype(jnp.int32)
    rcv = receivers.astype(jnp.int32)
    pad_g = jnp.zeros((epad - e,), jnp.int32)
    pad_s = jnp.full((epad - e,), n, jnp.int32)
    g_snd = _gather_perm(jnp.concatenate([snd, pad_g]))
    g_rcv = _gather_perm(jnp.concatenate([rcv, pad_g]))
    s_snd = _scatter_perm(jnp.concatenate([snd, pad_s])).reshape(-1, 128)
    s_rcv = _scatter_perm(jnp.concatenate([rcv, pad_s])).reshape(-1, 128)

    T = jnp.concatenate(
        [positions.astype(jnp.float32),
         species.astype(jnp.float32)[:, None],
         jnp.zeros((n, 4), jnp.float32)], axis=1)
    ts, tr = _sc_gather(T, g_snd, g_rcv)
    edge128 = _edge_dense(ts.reshape(epad // 16, 128),
                          tr.reshape(epad // 16, 128), W0, W1, Wl, Wout)
    edge = edge128.reshape(epad, 32)
    part = _sc_scatter(edge, s_snd, s_rcv,
                       jnp.zeros((nacc, edge.shape[1]), jnp.float32))
    return part[:n] + part[nacc:nacc + n]


# speedup vs baseline: 10.5843x; 1.1157x over previous
import functools
import math

import jax
import jax.numpy as jnp
from jax import lax
from jax.experimental import pallas as pl
from jax.experimental.pallas import tpu as pltpu
from jax.experimental.pallas import tpu_sc as plsc

R_MAX = 5.0
AVG_NEIGH = 32.0
N_RADIAL = 8
NUM_SPECIES = 5

NC = 2
NS = 16
NW = NC * NS
BE = 2048


def _gather_perm(a):
    return a.reshape(-1, 16, 128).swapaxes(1, 2).reshape(-1)


def _edge_row_pos(epad):
    ar = jnp.arange(epad, dtype=jnp.int32)
    tb = ar % BE
    return (ar - tb) + 4 * (tb % 512) + tb // 512


def _sc_gather(T, senders, receivers):
    n, d = T.shape
    e = senders.shape[0]
    ew = e // NW
    mesh = plsc.VectorSubcoreMesh(core_axis_name="c", subcore_axis_name="s")

    @functools.partial(
        pl.kernel,
        out_type=(jax.ShapeDtypeStruct((e, d), jnp.float32),
                  jax.ShapeDtypeStruct((e, d), jnp.float32)),
        mesh=mesh,
        scratch_types=[
            pltpu.VMEM((ew,), jnp.int32),
            pltpu.VMEM((ew, d), jnp.float32),
            pltpu.SemaphoreType.DMA,
        ],
        compiler_params=pltpu.CompilerParams(use_tc_tiling_on_sc=False),
    )
    def k(t_hbm, snd_hbm, rcv_hbm, outs_hbm, outr_hbm, idx_v, rows_v, sem):
        wid = lax.axis_index("s") * NC + lax.axis_index("c")
        base = wid * ew
        pltpu.sync_copy(snd_hbm.at[pl.ds(base, ew)], idx_v)
        pltpu.async_copy(t_hbm.at[idx_v], rows_v, sem).wait()
        pltpu.sync_copy(rows_v, outs_hbm.at[pl.ds(base, ew)])
        pltpu.sync_copy(rcv_hbm.at[pl.ds(base, ew)], idx_v)
        pltpu.async_copy(t_hbm.at[idx_v], rows_v, sem).wait()
        pltpu.sync_copy(rows_v, outr_hbm.at[pl.ds(base, ew)])

    return k(T, senders, receivers)


def _sc_scatter(edge, snd2, rcv2, qpos, zeros):
    e, dout = edge.shape
    nacc = zeros.shape[0]
    nchunk, ch = snd2.shape
    ew = e // NW
    cw = nchunk // NW
    eb = 2048
    nblk = ew // eb
    cpb = eb // ch
    nslice = nacc // NS
    mesh = plsc.VectorSubcoreMesh(core_axis_name="c", subcore_axis_name="s")

    @functools.partial(
        pl.kernel,
        out_type=jax.ShapeDtypeStruct((NC * nacc, dout), jnp.float32),
        mesh=mesh,
        scratch_types=[
            pltpu.VMEM_SHARED((nacc, dout), jnp.float32),
            pltpu.VMEM((cw, ch), jnp.int32),
            pltpu.VMEM((cw, ch), jnp.int32),
            pltpu.VMEM((ew,), jnp.int32),
            pltpu.VMEM((eb, dout), jnp.float32),
            pltpu.SemaphoreType.DMA,
        ],
        compiler_params=pltpu.CompilerParams(use_tc_tiling_on_sc=False),
    )
    def k(edge_hbm, snd_hbm, rcv_hbm, q_hbm, z_hbm, out_hbm,
          acc, sidx, ridx, qv, ebuf, sem):
        cid = lax.axis_index("c")
        sid = lax.axis_index("s")
        wid = sid * NC + cid
        pltpu.sync_copy(z_hbm.at[pl.ds(sid * nslice, nslice)],
                        acc.at[pl.ds(sid * nslice, nslice)])
        pltpu.sync_copy(snd_hbm.at[pl.ds(wid * cw, cw)], sidx)
        pltpu.sync_copy(rcv_hbm.at[pl.ds(wid * cw, cw)], ridx)
        pltpu.sync_copy(q_hbm.at[pl.ds(wid * ew, ew)], qv)
        plsc.subcore_barrier()
        for blk in range(nblk):
            pltpu.async_copy(edge_hbm.at[qv.at[pl.ds(blk * eb, eb)]],
                             ebuf, sem).wait()
            for j in range(cpb):
                c = blk * cpb + j
                sl = ebuf.at[pl.ds(j * ch, ch)]
                pltpu.sync_copy(sl, acc.at[ridx.at[c]], add=True)
                pltpu.sync_copy(sl, acc.at[sidx.at[c]], add=True)
        plsc.subcore_barrier()
        pltpu.sync_copy(acc.at[pl.ds(sid * nslice, nslice)],
                        out_hbm.at[pl.ds(cid * nacc + sid * nslice, nslice)])

    return k(edge, snd2, rcv2, qpos, zeros)


def _sinpoly(r):
    r2 = r * r
    return r * (1.0 + r2 * (-1.0 / 6.0 + r2 * (1.0 / 120.0
               + r2 * (-1.0 / 5040.0 + r2 * (1.0 / 362880.0)))))


def _dense_body(ts_ref, tr_ref, w0sT_ref, w0rT_ref, w0cT_ref, w1T_ref,
                wlaT_ref, wlbT_ref, woutT_ref, out_ref):
    xs = jnp.transpose(ts_ref[...])
    xr_t = jnp.transpose(tr_ref[...])
    tsT = jnp.concatenate([xs[8 * j:8 * j + 8, :] for j in range(16)],
                          axis=1)
    trT = jnp.concatenate([xr_t[8 * j:8 * j + 8, :] for j in range(16)],
                          axis=1)
    b = tsT.shape[1]
    relT = (trT[0:3] - tsT[0:3]) * (1.0 / R_MAX)
    r2 = jnp.sum(relT * relT, axis=0, keepdims=True)
    d2 = r2 + 1e-9
    d = jnp.sqrt(d2)
    xr = jnp.clip(d, 1e-4, 1.0)
    t = jnp.clip(d, 0.0, 1.0) - 0.5
    cutoff = 0.5 - 0.5 * _sinpoly(jnp.pi * t)
    nb = 1.0 + jax.lax.broadcasted_iota(
        jnp.int32, (N_RADIAL, b), 0).astype(jnp.float32)
    z = nb * (jnp.pi * xr)
    q = jnp.floor(z * (1.0 / jnp.pi) + 0.5)
    r = z - q * jnp.pi
    par = q * 0.5 - jnp.floor(q * 0.5)
    sign = 1.0 - 4.0 * par
    s = sign * _sinpoly(r)
    rbfT = (math.sqrt(2.0) * s) * (cutoff / xr)

    iota5 = jax.lax.broadcasted_iota(
        jnp.int32, (NUM_SPECIES, b), 0).astype(jnp.float32)
    ohsT = jnp.where(tsT[3:4] == iota5, 1.0, 0.0)
    ohrT = jnp.where(trT[3:4] == iota5, 1.0, 0.0)
    dot = lambda a, x: jnp.dot(a, x, preferred_element_type=jnp.float32)
    h = jax.nn.silu(dot(w0cT_ref[...], rbfT) + dot(w0sT_ref[...], ohsT)
                    + dot(w0rT_ref[...], ohrT))
    h = jax.nn.silu(dot(w1T_ref[...], h))

    ny = 1.0 + 3.0 * (r2 / d2)
    g = h * 0.5
    for i in range(wlaT_ref.shape[0]):
        inv2 = (g * g) * ny
        h = jax.nn.silu(dot(wlaT_ref[i], h) + dot(wlbT_ref[i], inv2))
        g = g * h * (1.0 / math.sqrt(AVG_NEIGH))

    eT = dot(woutT_ref[...], h) * cutoff
    y = jnp.concatenate([eT[:, 512 * k:512 * (k + 1)] for k in range(4)],
                        axis=0)
    out_ref[...] = jnp.transpose(y)


def _edge_dense(ts128, tr128, W0, W1, Wl, Wout, interpret=False):
    E = ts128.shape[0] * 16
    hidden = W1.shape[0]
    W0sT = W0[0:NUM_SPECIES].T
    W0rT = W0[NUM_SPECIES:2 * NUM_SPECIES].T
    W0cT = W0[2 * NUM_SPECIES:].T
    WlaT = jnp.swapaxes(Wl[:, :hidden, :], 1, 2)
    WlbT = jnp.swapaxes(Wl[:, hidden:, :], 1, 2)
    WoutT = Wout.T
    grid = (E // BE,)
    full = lambda s: pl.BlockSpec(s, lambda i: tuple(0 for _ in s))
    return pl.pallas_call(
        _dense_body,
        grid=grid,
        in_specs=[
            pl.BlockSpec((BE // 16, 128), lambda i: (i, 0)),
            pl.BlockSpec((BE // 16, 128), lambda i: (i, 0)),
            full(W0sT.shape), full(W0rT.shape), full(W0cT.shape),
            full(W1.T.shape), full(WlaT.shape), full(WlbT.shape),
            full(WoutT.shape),
        ],
        out_specs=pl.BlockSpec((BE // 4, 128), lambda i: (i, 0)),
        out_shape=jax.ShapeDtypeStruct((E // 4, 128), jnp.float32),
        interpret=interpret,
    )(ts128, tr128, W0sT, W0rT, W0cT, W1.T, WlaT, WlbT, WoutT)


def kernel(positions, species, senders, receivers, W0, W1, Wl, Wout):
    n = positions.shape[0]
    e = senders.shape[0]
    epad = -(-e // (NW * BE)) * (NW * BE)
    nacc = -(-(n + 1) // NS) * NS
    snd = senders.astype(jnp.int32)
    rcv = receivers.astype(jnp.int32)
    pad_g = jnp.zeros((epad - e,), jnp.int32)
    pad_s = jnp.full((epad - e,), n, jnp.int32)
    g_snd = _gather_perm(jnp.concatenate([snd, pad_g]))
    g_rcv = _gather_perm(jnp.concatenate([rcv, pad_g]))
    s_snd = jnp.concatenate([snd, pad_s]).reshape(-1, 128)
    s_rcv = jnp.concatenate([rcv, pad_s]).reshape(-1, 128)

    T = jnp.concatenate(
        [positions.astype(jnp.float32),
         species.astype(jnp.float32)[:, None],
         jnp.zeros((n, 4), jnp.float32)], axis=1)
    ts, tr = _sc_gather(T, g_snd, g_rcv)
    edge128 = _edge_dense(ts.reshape(epad // 16, 128),
                          tr.reshape(epad // 16, 128), W0, W1, Wl, Wout)
    edge = edge128.reshape(epad, 32)
    part = _sc_scatter(edge, s_snd, s_rcv, _edge_row_pos(epad),
                       jnp.zeros((nacc, edge.shape[1]), jnp.float32))
    return part[:n] + part[nacc:nacc + n]


# speedup vs baseline: 10.9897x; 1.0383x over previous
import functools
import math

import jax
import jax.numpy as jnp
from jax import lax
from jax.experimental import pallas as pl
from jax.experimental.pallas import tpu as pltpu
from jax.experimental.pallas import tpu_sc as plsc

R_MAX = 5.0
AVG_NEIGH = 32.0
N_RADIAL = 8
NUM_SPECIES = 5

NC = 2
NS = 16
NW = NC * NS
BE = 2048


def _gather_perm(a):
    return a.reshape(-1, 16, 128).swapaxes(1, 2).reshape(-1)


def _edge_row_pos(epad):
    ar = jnp.arange(epad, dtype=jnp.int32)
    tb = ar % BE
    return (ar - tb) + 4 * (tb % 512) + tb // 512


def _sc_gather(T, senders, receivers):
    n, d = T.shape
    e = senders.shape[0]
    ew = e // NW
    eb = 2048
    nblk = ew // eb
    mesh = plsc.VectorSubcoreMesh(core_axis_name="c", subcore_axis_name="s")

    @functools.partial(
        pl.kernel,
        out_type=(jax.ShapeDtypeStruct((e, d), jnp.float32),
                  jax.ShapeDtypeStruct((e, d), jnp.float32)),
        mesh=mesh,
        scratch_types=[
            pltpu.VMEM((ew,), jnp.int32),
            pltpu.VMEM((ew,), jnp.int32),
            pltpu.VMEM((2, eb, d), jnp.float32),
            pltpu.SemaphoreType.DMA,
            pltpu.SemaphoreType.DMA,
        ],
        compiler_params=pltpu.CompilerParams(use_tc_tiling_on_sc=False),
    )
    def k(t_hbm, snd_hbm, rcv_hbm, outs_hbm, outr_hbm,
          idxs, idxr, rows, sem0, sem1):
        wid = lax.axis_index("s") * NC + lax.axis_index("c")
        base = wid * ew
        pltpu.sync_copy(snd_hbm.at[pl.ds(base, ew)], idxs)
        pltpu.sync_copy(rcv_hbm.at[pl.ds(base, ew)], idxr)
        tasks = [(idxs, outs_hbm, b) for b in range(nblk)] \
              + [(idxr, outr_hbm, b) for b in range(nblk)]
        sems = (sem0, sem1)

        def issue(t):
            idx_ref, _, b = tasks[t]
            return pltpu.async_copy(
                t_hbm.at[idx_ref.at[pl.ds(b * eb, eb)]],
                rows.at[t % 2], sems[t % 2])

        cps = [issue(0), issue(1)]
        for t in range(len(tasks)):
            _, out_hbm, b = tasks[t]
            cps[t % 2].wait()
            pltpu.sync_copy(rows.at[t % 2],
                            out_hbm.at[pl.ds(base + b * eb, eb)])
            if t + 2 < len(tasks):
                cps[t % 2] = issue(t + 2)

    return k(T, senders, receivers)


def _sc_scatter(edge, snd2, rcv2, qpos, zeros):
    e, dout = edge.shape
    nacc = zeros.shape[0]
    nchunk, ch = snd2.shape
    ew = e // NW
    cw = nchunk // NW
    eb = 1024
    nblk = ew // eb
    cpb = eb // ch
    nslice = nacc // NS
    mesh = plsc.VectorSubcoreMesh(core_axis_name="c", subcore_axis_name="s")

    @functools.partial(
        pl.kernel,
        out_type=jax.ShapeDtypeStruct((NC * nacc, dout), jnp.float32),
        mesh=mesh,
        scratch_types=[
            pltpu.VMEM_SHARED((nacc, dout), jnp.float32),
            pltpu.VMEM((cw, ch), jnp.int32),
            pltpu.VMEM((cw, ch), jnp.int32),
            pltpu.VMEM((ew,), jnp.int32),
            pltpu.VMEM((2, eb, dout), jnp.float32),
            pltpu.SemaphoreType.DMA,
            pltpu.SemaphoreType.DMA,
            pltpu.SemaphoreType.DMA,
        ],
        compiler_params=pltpu.CompilerParams(use_tc_tiling_on_sc=False),
    )
    def k(edge_hbm, snd_hbm, rcv_hbm, q_hbm, z_hbm, out_hbm,
          acc, sidx, ridx, qv, ebuf, gsem0, gsem1, ssem):
        cid = lax.axis_index("c")
        sid = lax.axis_index("s")
        wid = sid * NC + cid
        pltpu.sync_copy(z_hbm.at[pl.ds(sid * nslice, nslice)],
                        acc.at[pl.ds(sid * nslice, nslice)])
        pltpu.sync_copy(snd_hbm.at[pl.ds(wid * cw, cw)], sidx)
        pltpu.sync_copy(rcv_hbm.at[pl.ds(wid * cw, cw)], ridx)
        pltpu.sync_copy(q_hbm.at[pl.ds(wid * ew, ew)], qv)
        plsc.subcore_barrier()
        gsems = (gsem0, gsem1)

        def gissue(b):
            return pltpu.async_copy(
                edge_hbm.at[qv.at[pl.ds(b * eb, eb)]],
                ebuf.at[b % 2], gsems[b % 2])

        cps = [gissue(0), None]
        prev_sc = []
        for blk in range(nblk):
            slot = blk % 2
            for c_ in prev_sc:
                c_.wait()
            prev_sc = []
            if blk + 1 < nblk:
                cps[1 - slot] = gissue(blk + 1)
            cps[slot].wait()
            for j in range(cpb):
                c = blk * cpb + j
                sl = ebuf.at[slot].at[pl.ds(j * ch, ch)]
                prev_sc.append(
                    pltpu.async_copy(sl, acc.at[ridx.at[c]], ssem, add=True))
                prev_sc.append(
                    pltpu.async_copy(sl, acc.at[sidx.at[c]], ssem, add=True))
        for c_ in prev_sc:
            c_.wait()
        plsc.subcore_barrier()
        pltpu.sync_copy(acc.at[pl.ds(sid * nslice, nslice)],
                        out_hbm.at[pl.ds(cid * nacc + sid * nslice, nslice)])

    return k(edge, snd2, rcv2, qpos, zeros)


def _sinpoly(r):
    r2 = r * r
    return r * (1.0 + r2 * (-1.0 / 6.0 + r2 * (1.0 / 120.0
               + r2 * (-1.0 / 5040.0 + r2 * (1.0 / 362880.0)))))


def _dense_body(ts_ref, tr_ref, w0sT_ref, w0rT_ref, w0cT_ref, w1T_ref,
                wlaT_ref, wlbT_ref, woutT_ref, out_ref):
    xs = jnp.transpose(ts_ref[...])
    xr_t = jnp.transpose(tr_ref[...])
    tsT = jnp.concatenate([xs[8 * j:8 * j + 8, :] for j in range(16)],
                          axis=1)
    trT = jnp.concatenate([xr_t[8 * j:8 * j + 8, :] for j in range(16)],
                          axis=1)
    b = tsT.shape[1]
    relT = (trT[0:3] - tsT[0:3]) * (1.0 / R_MAX)
    r2 = jnp.sum(relT * relT, axis=0, keepdims=True)
    d2 = r2 + 1e-9
    d = jnp.sqrt(d2)
    xr = jnp.clip(d, 1e-4, 1.0)
    t = jnp.clip(d, 0.0, 1.0) - 0.5
    cutoff = 0.5 - 0.5 * _sinpoly(jnp.pi * t)
    nb = 1.0 + jax.lax.broadcasted_iota(
        jnp.int32, (N_RADIAL, b), 0).astype(jnp.float32)
    z = nb * (jnp.pi * xr)
    q = jnp.floor(z * (1.0 / jnp.pi) + 0.5)
    r = z - q * jnp.pi
    par = q * 0.5 - jnp.floor(q * 0.5)
    sign = 1.0 - 4.0 * par
    s = sign * _sinpoly(r)
    rbfT = (math.sqrt(2.0) * s) * (cutoff / xr)

    iota5 = jax.lax.broadcasted_iota(
        jnp.int32, (NUM_SPECIES, b), 0).astype(jnp.float32)
    ohsT = jnp.where(tsT[3:4] == iota5, 1.0, 0.0)
    ohrT = jnp.where(trT[3:4] == iota5, 1.0, 0.0)
    dot = lambda a, x: jnp.dot(a, x, preferred_element_type=jnp.float32)
    h = jax.nn.silu(dot(w0cT_ref[...], rbfT) + dot(w0sT_ref[...], ohsT)
                    + dot(w0rT_ref[...], ohrT))
    h = jax.nn.silu(dot(w1T_ref[...], h))

    ny = 1.0 + 3.0 * (r2 / d2)
    g = h * 0.5
    for i in range(wlaT_ref.shape[0]):
        inv2 = (g * g) * ny
        h = jax.nn.silu(dot(wlaT_ref[i], h) + dot(wlbT_ref[i], inv2))
        g = g * h * (1.0 / math.sqrt(AVG_NEIGH))

    eT = dot(woutT_ref[...], h) * cutoff
    y = jnp.concatenate([eT[:, 512 * k:512 * (k + 1)] for k in range(4)],
                        axis=0)
    out_ref[...] = jnp.transpose(y)


def _edge_dense(ts128, tr128, W0, W1, Wl, Wout, interpret=False):
    E = ts128.shape[0] * 16
    hidden = W1.shape[0]
    W0sT = W0[0:NUM_SPECIES].T
    W0rT = W0[NUM_SPECIES:2 * NUM_SPECIES].T
    W0cT = W0[2 * NUM_SPECIES:].T
    WlaT = jnp.swapaxes(Wl[:, :hidden, :], 1, 2)
    WlbT = jnp.swapaxes(Wl[:, hidden:, :], 1, 2)
    WoutT = Wout.T
    grid = (E // BE,)
    full = lambda s: pl.BlockSpec(s, lambda i: tuple(0 for _ in s))
    return pl.pallas_call(
        _dense_body,
        grid=grid,
        in_specs=[
            pl.BlockSpec((BE // 16, 128), lambda i: (i, 0)),
            pl.BlockSpec((BE // 16, 128), lambda i: (i, 0)),
            full(W0sT.shape), full(W0rT.shape), full(W0cT.shape),
            full(W1.T.shape), full(WlaT.shape), full(WlbT.shape),
            full(WoutT.shape),
        ],
        out_specs=pl.BlockSpec((BE // 4, 128), lambda i: (i, 0)),
        out_shape=jax.ShapeDtypeStruct((E // 4, 128), jnp.float32),
        interpret=interpret,
    )(ts128, tr128, W0sT, W0rT, W0cT, W1.T, WlaT, WlbT, WoutT)


def kernel(positions, species, senders, receivers, W0, W1, Wl, Wout):
    n = positions.shape[0]
    e = senders.shape[0]
    epad = -(-e // (NW * BE)) * (NW * BE)
    nacc = -(-(n + 1) // NS) * NS
    snd = senders.astype(jnp.int32)
    rcv = receivers.astype(jnp.int32)
    pad_g = jnp.zeros((epad - e,), jnp.int32)
    pad_s = jnp.full((epad - e,), n, jnp.int32)
    g_snd = _gather_perm(jnp.concatenate([snd, pad_g]))
    g_rcv = _gather_perm(jnp.concatenate([rcv, pad_g]))
    s_snd = jnp.concatenate([snd, pad_s]).reshape(-1, 128)
    s_rcv = jnp.concatenate([rcv, pad_s]).reshape(-1, 128)

    T = jnp.concatenate(
        [positions.astype(jnp.float32),
         species.astype(jnp.float32)[:, None],
         jnp.zeros((n, 4), jnp.float32)], axis=1)
    ts, tr = _sc_gather(T, g_snd, g_rcv)
    edge128 = _edge_dense(ts.reshape(epad // 16, 128),
                          tr.reshape(epad // 16, 128), W0, W1, Wl, Wout)
    edge = edge128.reshape(epad, 32)
    part = _sc_scatter(edge, s_snd, s_rcv, _edge_row_pos(epad),
                       jnp.zeros((nacc, edge.shape[1]), jnp.float32))
    return part[:n] + part[nacc:nacc + n]


# speedup vs baseline: 11.7491x; 1.0691x over previous
import functools
import math

import jax
import jax.numpy as jnp
from jax import lax
from jax.experimental import pallas as pl
from jax.experimental.pallas import tpu as pltpu
from jax.experimental.pallas import tpu_sc as plsc

R_MAX = 5.0
AVG_NEIGH = 32.0
N_RADIAL = 8
NUM_SPECIES = 5

NC = 2
NS = 16
NW = NC * NS
BE = 2048


def _gather_perm(a):
    return a.reshape(-1, 16, 128).swapaxes(1, 2).reshape(-1)


def _edge_row_pos(epad):
    ar = jnp.arange(epad, dtype=jnp.int32)
    tb = ar % BE
    return (ar - tb) + 4 * (tb % 512) + tb // 512


def _sc_gather(T, senders, receivers):
    n, d = T.shape
    e = senders.shape[0]
    ew = e // NW
    eb = 2048
    nblk = ew // eb
    mesh = plsc.VectorSubcoreMesh(core_axis_name="c", subcore_axis_name="s")

    @functools.partial(
        pl.kernel,
        out_type=(jax.ShapeDtypeStruct((e, d), jnp.float32),
                  jax.ShapeDtypeStruct((e, d), jnp.float32)),
        mesh=mesh,
        scratch_types=[
            pltpu.VMEM((ew,), jnp.int32),
            pltpu.VMEM((ew,), jnp.int32),
            pltpu.VMEM((2, eb, d), jnp.float32),
            pltpu.SemaphoreType.DMA,
            pltpu.SemaphoreType.DMA,
        ],
        compiler_params=pltpu.CompilerParams(use_tc_tiling_on_sc=False),
    )
    def k(t_hbm, snd_hbm, rcv_hbm, outs_hbm, outr_hbm,
          idxs, idxr, rows, sem0, sem1):
        wid = lax.axis_index("s") * NC + lax.axis_index("c")
        base = wid * ew
        pltpu.sync_copy(snd_hbm.at[pl.ds(base, ew)], idxs)
        pltpu.sync_copy(rcv_hbm.at[pl.ds(base, ew)], idxr)
        tasks = [(idxs, outs_hbm, b) for b in range(nblk)] \
              + [(idxr, outr_hbm, b) for b in range(nblk)]
        sems = (sem0, sem1)

        def issue(t):
            idx_ref, _, b = tasks[t]
            return pltpu.async_copy(
                t_hbm.at[idx_ref.at[pl.ds(b * eb, eb)]],
                rows.at[t % 2], sems[t % 2])

        cps = [issue(0), issue(1)]
        for t in range(len(tasks)):
            _, out_hbm, b = tasks[t]
            cps[t % 2].wait()
            pltpu.sync_copy(rows.at[t % 2],
                            out_hbm.at[pl.ds(base + b * eb, eb)])
            if t + 2 < len(tasks):
                cps[t % 2] = issue(t + 2)

    return k(T, senders, receivers)


def _sc_scatter(edge, snd2, rcv2, qpos, zeros):
    e, dout = edge.shape
    nacc = zeros.shape[0]
    nchunk, ch = snd2.shape
    ew = e // NW
    cw = nchunk // NW
    eb = 1024
    nblk = ew // eb
    cpb = eb // ch
    nslice = nacc // NS
    mesh = plsc.VectorSubcoreMesh(core_axis_name="c", subcore_axis_name="s")

    @functools.partial(
        pl.kernel,
        out_type=jax.ShapeDtypeStruct((NC * nacc, dout), jnp.float32),
        mesh=mesh,
        scratch_types=[
            pltpu.VMEM_SHARED((nacc, dout), jnp.float32),
            pltpu.VMEM((cw, ch), jnp.int32),
            pltpu.VMEM((cw, ch), jnp.int32),
            pltpu.VMEM((ew,), jnp.int32),
            pltpu.VMEM((2, eb, dout), jnp.float32),
            pltpu.SemaphoreType.DMA,
            pltpu.SemaphoreType.DMA,
            pltpu.SemaphoreType.DMA,
        ],
        compiler_params=pltpu.CompilerParams(use_tc_tiling_on_sc=False),
    )
    def k(edge_hbm, snd_hbm, rcv_hbm, q_hbm, z_hbm, out_hbm,
          acc, sidx, ridx, qv, ebuf, gsem0, gsem1, ssem):
        cid = lax.axis_index("c")
        sid = lax.axis_index("s")
        wid = sid * NC + cid
        pltpu.sync_copy(z_hbm.at[pl.ds(sid * nslice, nslice)],
                        acc.at[pl.ds(sid * nslice, nslice)])
        pltpu.sync_copy(snd_hbm.at[pl.ds(wid * cw, cw)], sidx)
        pltpu.sync_copy(rcv_hbm.at[pl.ds(wid * cw, cw)], ridx)
        pltpu.sync_copy(q_hbm.at[pl.ds(wid * ew, ew)], qv)
        plsc.subcore_barrier()
        gsems = (gsem0, gsem1)

        def gissue(b):
            return pltpu.async_copy(
                edge_hbm.at[qv.at[pl.ds(b * eb, eb)]],
                ebuf.at[b % 2], gsems[b % 2])

        cps = [gissue(0), None]
        prev_sc = []
        for blk in range(nblk):
            slot = blk % 2
            for c_ in prev_sc:
                c_.wait()
            prev_sc = []
            if blk + 1 < nblk:
                cps[1 - slot] = gissue(blk + 1)
            cps[slot].wait()
            for j in range(cpb):
                c = blk * cpb + j
                sl = ebuf.at[slot].at[pl.ds(j * ch, ch)]
                prev_sc.append(
                    pltpu.async_copy(sl, acc.at[ridx.at[c]], ssem, add=True))
                prev_sc.append(
                    pltpu.async_copy(sl, acc.at[sidx.at[c]], ssem, add=True))
        for c_ in prev_sc:
            c_.wait()
        plsc.subcore_barrier()
        pltpu.sync_copy(acc.at[pl.ds(sid * nslice, nslice)],
                        out_hbm.at[pl.ds(cid * nacc + sid * nslice, nslice)])

    return k(edge, snd2, rcv2, qpos, zeros)


def _sinpoly(r):
    r2 = r * r
    return r * (1.0 + r2 * (-1.0 / 6.0 + r2 * (1.0 / 120.0
               + r2 * (-1.0 / 5040.0 + r2 * (1.0 / 362880.0)))))


def _dense_half(ts_ref, tr_ref, w0sT_ref, w0rT_ref, w0cT_ref, w1T_ref,
                wlaT_ref, wlbT_ref, woutT_ref, out_ref, half):
    xs = jnp.transpose(ts_ref[128 * half:128 * half + 128, :])
    xr_t = jnp.transpose(tr_ref[128 * half:128 * half + 128, :])
    tsT = jnp.concatenate([xs[8 * j:8 * j + 8, :] for j in range(16)],
                          axis=1)
    trT = jnp.concatenate([xr_t[8 * j:8 * j + 8, :] for j in range(16)],
                          axis=1)
    b = tsT.shape[1]
    relT = (trT[0:3] - tsT[0:3]) * (1.0 / R_MAX)
    r2 = jnp.sum(relT * relT, axis=0, keepdims=True)
    d2 = r2 + 1e-9
    d = jnp.sqrt(d2)
    xr = jnp.clip(d, 1e-4, 1.0)
    t = jnp.clip(d, 0.0, 1.0) - 0.5
    cutoff = 0.5 - 0.5 * _sinpoly(jnp.pi * t)
    nb = 1.0 + jax.lax.broadcasted_iota(
        jnp.int32, (N_RADIAL, b), 0).astype(jnp.float32)
    z = nb * (jnp.pi * xr)
    q = jnp.floor(z * (1.0 / jnp.pi) + 0.5)
    r = z - q * jnp.pi
    par = q * 0.5 - jnp.floor(q * 0.5)
    sign = 1.0 - 4.0 * par
    s = sign * _sinpoly(r)
    rbfT = (math.sqrt(2.0) * s) * (cutoff / xr)

    iota5 = jax.lax.broadcasted_iota(
        jnp.int32, (NUM_SPECIES, b), 0).astype(jnp.float32)
    ohsT = jnp.where(tsT[3:4] == iota5, 1.0, 0.0)
    ohrT = jnp.where(trT[3:4] == iota5, 1.0, 0.0)
    dot = lambda a, x: jnp.dot(a, x, preferred_element_type=jnp.float32)
    h = jax.nn.silu(dot(w0cT_ref[...], rbfT) + dot(w0sT_ref[...], ohsT)
                    + dot(w0rT_ref[...], ohrT))
    h = jax.nn.silu(dot(w1T_ref[...], h))

    ny = 1.0 + 3.0 * (r2 / d2)
    g = h * 0.5
    for i in range(wlaT_ref.shape[0]):
        inv2 = (g * g) * ny
        h = jax.nn.silu(dot(wlaT_ref[i], h) + dot(wlbT_ref[i], inv2))
        g = g * h * (1.0 / math.sqrt(AVG_NEIGH))

    eT = dot(woutT_ref[...], h) * cutoff
    y = jnp.concatenate([eT[:, 512 * k:512 * (k + 1)] for k in range(4)],
                        axis=0)
    out_ref[512 * half:512 * half + 512, :] = jnp.transpose(y)


def _dense_body(*refs):
    _dense_half(*refs, 0)
    _dense_half(*refs, 1)


def _edge_dense(ts128, tr128, W0, W1, Wl, Wout, interpret=False):
    E = ts128.shape[0] * 16
    hidden = W1.shape[0]
    W0sT = W0[0:NUM_SPECIES].T
    W0rT = W0[NUM_SPECIES:2 * NUM_SPECIES].T
    W0cT = W0[2 * NUM_SPECIES:].T
    WlaT = jnp.swapaxes(Wl[:, :hidden, :], 1, 2)
    WlbT = jnp.swapaxes(Wl[:, hidden:, :], 1, 2)
    WoutT = Wout.T
    grid = (E // (2 * BE),)
    full = lambda s: pl.BlockSpec(s, lambda i: tuple(0 for _ in s))
    return pl.pallas_call(
        _dense_body,
        grid=grid,
        in_specs=[
            pl.BlockSpec((2 * BE // 16, 128), lambda i: (i, 0)),
            pl.BlockSpec((2 * BE // 16, 128), lambda i: (i, 0)),
            full(W0sT.shape), full(W0rT.shape), full(W0cT.shape),
            full(W1.T.shape), full(WlaT.shape), full(WlbT.shape),
            full(WoutT.shape),
        ],
        out_specs=pl.BlockSpec((2 * BE // 4, 128), lambda i: (i, 0)),
        out_shape=jax.ShapeDtypeStruct((E // 4, 128), jnp.float32),
        interpret=interpret,
    )(ts128, tr128, W0sT, W0rT, W0cT, W1.T, WlaT, WlbT, WoutT)


def kernel(positions, species, senders, receivers, W0, W1, Wl, Wout):
    n = positions.shape[0]
    e = senders.shape[0]
    epad = -(-e // (NW * BE)) * (NW * BE)
    nacc = -(-(n + 1) // NS) * NS
    snd = senders.astype(jnp.int32)
    rcv = receivers.astype(jnp.int32)
    pad_g = jnp.zeros((epad - e,), jnp.int32)
    pad_s = jnp.full((epad - e,), n, jnp.int32)
    g_snd = _gather_perm(jnp.concatenate([snd, pad_g]))
    g_rcv = _gather_perm(jnp.concatenate([rcv, pad_g]))
    s_snd = jnp.concatenate([snd, pad_s]).reshape(-1, 128)
    s_rcv = jnp.concatenate([rcv, pad_s]).reshape(-1, 128)

    T = jnp.concatenate(
        [positions.astype(jnp.float32),
         species.astype(jnp.float32)[:, None],
         jnp.zeros((n, 4), jnp.float32)], axis=1)
    ts, tr = _sc_gather(T, g_snd, g_rcv)
    edge128 = _edge_dense(ts.reshape(epad // 16, 128),
                          tr.reshape(epad // 16, 128), W0, W1, Wl, Wout)
    edge = edge128.reshape(epad, 32)
    part = _sc_scatter(edge, s_snd, s_rcv, _edge_row_pos(epad),
                       jnp.zeros((nacc, edge.shape[1]), jnp.float32))
    return part[:n] + part[nacc:nacc + n]
